# Initial kernel scaffold; baseline (speedup 1.0000x reference)
import jax
import jax.numpy as jnp
from jax.experimental import pallas as pl


def kernel(x, edge_index, W, att, bias):
    raise NotImplementedError("write your pallas kernel here")


# speedup vs baseline: 1.2068x; 1.2068x over previous
import jax
import jax.numpy as jnp
from jax.experimental import pallas as pl


N_NODES = 10000
D = 128


def _mm_body(x_ref, w_ref, av_ref, xt_ref, a2_ref):
    xt = jnp.dot(x_ref[...], w_ref[...], preferred_element_type=jnp.float32)
    xt_ref[...] = xt
    a2_ref[...] = jax.lax.dot_general(
        av_ref[...], xt,
        dimension_numbers=(((0,), (1,)), ((), ())),
        preferred_element_type=jnp.float32,
    )


def _i32(v):
    return jnp.asarray(v, jnp.int32)


def _linear(xp, W, av, n_pad):
    blk = 512
    grid = (n_pad // blk,)
    z = lambda i: (_i32(0), _i32(0))
    return pl.pallas_call(
        _mm_body,
        grid=grid,
        in_specs=[
            pl.BlockSpec((blk, D), lambda i: (i, _i32(0))),
            pl.BlockSpec((D, D), z),
            pl.BlockSpec((D, 2), z),
        ],
        out_specs=[
            pl.BlockSpec((blk, D), lambda i: (i, _i32(0))),
            pl.BlockSpec((2, blk), lambda i: (_i32(0), i)),
        ],
        out_shape=[
            jax.ShapeDtypeStruct((n_pad, D), jnp.float32),
            jax.ShapeDtypeStruct((2, n_pad), jnp.float32),
        ],
    )(xp, W, av)


def kernel(x, edge_index, W, att, bias):
    N = x.shape[0]
    n_pad = 10240
    xp = jnp.zeros((n_pad, D), jnp.float32).at[:N].set(x.astype(jnp.float32))
    att_f = att.reshape(2 * D).astype(jnp.float32)
    av = jnp.stack([att_f[:D], att_f[D:]], axis=1)

    xt_pad, a2 = _linear(xp, W.astype(jnp.float32), av, n_pad)
    xt = xt_pad[:N]
    a_dst, a_src = a2[0], a2[1]

    row = edge_index[0].astype(jnp.int32)
    col = edge_index[1].astype(jnp.int32)
    loop = jnp.arange(N, dtype=jnp.int32)
    seg = jnp.concatenate([jnp.where(row != col, row, N), loop])
    colg = jnp.concatenate([col, loop])

    alpha = a_dst[jnp.minimum(seg, N - 1)] + a_src[colg]
    alpha = jnp.where(seg == N, 0.0, alpha)
    alpha = jnp.where(alpha > 0, alpha, 0.2 * alpha)
    p = jnp.exp(alpha)
    denom = jax.ops.segment_sum(p, seg, num_segments=N + 1)
    num = jax.ops.segment_sum(xt[colg] * p[:, None], seg, num_segments=N + 1)
    out = num[:N] / (denom[:N, None] + 1e-16) + bias
    return out.astype(jnp.float32)


# speedup vs baseline: 18.4806x; 15.3132x over previous
import dataclasses
import functools

import jax
import jax.numpy as jnp
from jax import lax
from jax.experimental import pallas as pl
from jax.experimental.pallas import tpu as pltpu
from jax.experimental.pallas import tpu_sc as plsc

N_NODES = 10000
D = 128
NP = 10240
NC = 2
NS = 16
L = 16
G = 128
K = 81
EP = NC * NS * G * K
RZ = NP // NS


def _i32(v):
    return jnp.asarray(v, jnp.int32)


def _mm_body(x_ref, w_ref, av_ref, xt_ref, a2_ref):
    xt = jnp.dot(x_ref[...], w_ref[...], preferred_element_type=jnp.float32)
    xt_ref[...] = xt
    a2_ref[...] = lax.dot_general(
        av_ref[...], xt,
        dimension_numbers=(((0,), (1,)), ((), ())),
        preferred_element_type=jnp.float32,
    )


def _linear(xp, W, av):
    blk = 512
    z = lambda i: (_i32(0), _i32(0))
    return pl.pallas_call(
        _mm_body,
        grid=(NP // blk,),
        in_specs=[
            pl.BlockSpec((blk, D), lambda i: (i, _i32(0))),
            pl.BlockSpec((D, D), z),
            pl.BlockSpec((D, 2), z),
        ],
        out_specs=[
            pl.BlockSpec((blk, D), lambda i: (i, _i32(0))),
            pl.BlockSpec((2, blk), lambda i: (_i32(0), i)),
        ],
        out_shape=[
            jax.ShapeDtypeStruct((NP, D), jnp.float32),
            jax.ShapeDtypeStruct((2, NP), jnp.float32),
        ],
    )(xp, W, av)


def _sc_edge(xt_pad, a_dst, a_src, seg, colg):
    mesh = plsc.VectorSubcoreMesh(core_axis_name="c", subcore_axis_name="s")
    cp = pltpu.CompilerParams()
    if "needs_layout_passes" in pltpu.CompilerParams.__dataclass_fields__:
        cp = dataclasses.replace(cp, needs_layout_passes=False)

    @functools.partial(
        pl.kernel,
        compiler_params=cp,
        out_type=[
            jax.ShapeDtypeStruct((NC, NP, D), jnp.float32),
            jax.ShapeDtypeStruct((NC * NS, NP), jnp.float32),
        ],
        mesh=mesh,
        scratch_types=[
            pltpu.VMEM((NP,), jnp.float32),
            pltpu.VMEM((NP,), jnp.float32),
            pltpu.VMEM((NP,), jnp.float32),
            pltpu.VMEM((G,), jnp.int32),
            pltpu.VMEM((G,), jnp.int32),
            pltpu.VMEM((G, D), jnp.float32),
            pltpu.VMEM((G,), jnp.float32),
            pltpu.VMEM_SHARED((NP, D), jnp.float32),
            pltpu.SemaphoreType.DMA,
        ],
    )
    def k(xt_hbm, adst_hbm, asrc_hbm, seg_hbm, col_hbm, num_hbm, den_hbm,
          adst_v, asrc_v, den_v, seg_v, col_v, rows_v, p_v, acc_sh, sem):
        c = lax.axis_index("c")
        s = lax.axis_index("s")
        wid = s * _i32(NC) + c
        z16 = jnp.zeros((L,), jnp.float32)

        @pl.loop(_i32(0), _i32(G))
        def _(j):
            for cc in range(D // L):
                rows_v[j, pl.ds(cc * L, L)] = z16

        for t in range(RZ // G):
            pltpu.sync_copy(rows_v, acc_sh.at[pl.ds(s * _i32(RZ) + _i32(t * G), G)])

        @pl.loop(_i32(0), _i32(NP // L))
        def _(i):
            den_v[pl.ds(i * _i32(L), L)] = z16

        pltpu.sync_copy(adst_hbm, adst_v)
        pltpu.sync_copy(asrc_hbm, asrc_v)

        plsc.subcore_barrier()

        @pl.loop(_i32(0), _i32(K))
        def _(kw):
            base = wid * _i32(G * K) + kw * _i32(G)
            pltpu.sync_copy(seg_hbm.at[pl.ds(base, G)], seg_v)
            pltpu.sync_copy(col_hbm.at[pl.ds(base, G)], col_v)
            pltpu.async_copy(xt_hbm.at[col_v], rows_v, sem).wait()

            for j8 in range(G // L):
                sidx = seg_v[pl.ds(j8 * L, L)]
                cidx = col_v[pl.ds(j8 * L, L)]
                al = (plsc.load_gather(adst_v, [sidx])
                      + plsc.load_gather(asrc_v, [cidx]))
                al = jnp.where(al > 0, al, al * 0.2)
                p = jnp.exp(al)
                p_v[pl.ds(j8 * L, L)] = p
                plsc.addupdate_scatter(den_v, [sidx], p)

            @pl.loop(_i32(0), _i32(G // L))
            def _(j16):
                jb = j16 * _i32(L)
                pvec = p_v[pl.ds(jb, L)]
                for l in range(L):
                    pv = jnp.broadcast_to(pvec[l], (L,))
                    for cc in range(D // L):
                        sl = pl.ds(cc * L, L)
                        rows_v[jb + _i32(l), sl] = rows_v[jb + _i32(l), sl] * pv

            pltpu.sync_copy(rows_v, acc_sh.at[seg_v], add=True)

        plsc.subcore_barrier()

        pltpu.sync_copy(acc_sh.at[pl.ds(s * _i32(RZ), RZ)],
                        num_hbm.at[c, pl.ds(s * _i32(RZ), RZ)])
        pltpu.sync_copy(den_v, den_hbm.at[wid])

    return k(xt_pad, a_dst, a_src, seg, colg)


def _combine_body(num_ref, den_ref, bias_ref, out_ref):
    n = num_ref[0] + num_ref[1]
    d = jnp.sum(den_ref[...], axis=0)
    out_ref[...] = n / (d[:, None] + 1e-16) + bias_ref[0][None, :]


def _combine(num, den, bias2d):
    blk = 512
    return pl.pallas_call(
        _combine_body,
        grid=(NP // blk,),
        in_specs=[
            pl.BlockSpec((NC, blk, D), lambda i: (_i32(0), i, _i32(0))),
            pl.BlockSpec((NC * NS, blk), lambda i: (_i32(0), i)),
            pl.BlockSpec((1, D), lambda i: (_i32(0), _i32(0))),
        ],
        out_specs=pl.BlockSpec((blk, D), lambda i: (i, _i32(0))),
        out_shape=jax.ShapeDtypeStruct((NP, D), jnp.float32),
    )(num, den, bias2d)


def kernel(x, edge_index, W, att, bias):
    N = x.shape[0]
    E = edge_index.shape[1]
    xp = jnp.zeros((NP, D), jnp.float32).at[:N].set(x.astype(jnp.float32))
    att_f = att.reshape(2 * D).astype(jnp.float32)
    av = jnp.stack([att_f[:D], att_f[D:]], axis=1)

    xt_pad, a2 = _linear(xp, W.astype(jnp.float32), av)
    a_dst, a_src = a2[0], a2[1]

    row = edge_index[0].astype(jnp.int32)
    col = edge_index[1].astype(jnp.int32)
    loop = jnp.arange(N, dtype=jnp.int32)
    pad = EP - E - N
    seg = jnp.concatenate([
        jnp.where(row != col, row, N), loop,
        jnp.full((pad,), N, jnp.int32),
    ])
    colg = jnp.concatenate([col, loop, jnp.zeros((pad,), jnp.int32)])

    num, den = _sc_edge(xt_pad, a_dst, a_src, seg, colg)
    out = _combine(num, den, bias.astype(jnp.float32).reshape(1, D))
    return out[:N]


# speedup vs baseline: 26.0982x; 1.4122x over previous
import dataclasses
import functools

import jax
import jax.numpy as jnp
from jax import lax
from jax.experimental import pallas as pl
from jax.experimental.pallas import tpu as pltpu
from jax.experimental.pallas import tpu_sc as plsc

N_NODES = 10000
D = 128
NP = 10240
NC = 2
NS = 16
L = 16
G = 64
K = 162
KW = K * G
EP = NC * NS * KW
RZ = NP // NS


def _i32(v):
    return jnp.asarray(v, jnp.int32)


def _mm_body(x_ref, w_ref, av_ref, xt_ref, a2_ref):
    xt = jnp.dot(x_ref[...], w_ref[...], preferred_element_type=jnp.float32)
    xt_ref[...] = xt
    a2_ref[...] = lax.dot_general(
        av_ref[...], xt,
        dimension_numbers=(((0,), (1,)), ((), ())),
        preferred_element_type=jnp.float32,
    )


def _linear(xp, W, av):
    blk = 512
    z = lambda i: (_i32(0), _i32(0))
    return pl.pallas_call(
        _mm_body,
        grid=(NP // blk,),
        in_specs=[
            pl.BlockSpec((blk, D), lambda i: (i, _i32(0))),
            pl.BlockSpec((D, D), z),
            pl.BlockSpec((D, 2), z),
        ],
        out_specs=[
            pl.BlockSpec((blk, D), lambda i: (i, _i32(0))),
            pl.BlockSpec((2, blk), lambda i: (_i32(0), i)),
        ],
        out_shape=[
            jax.ShapeDtypeStruct((NP, D), jnp.float32),
            jax.ShapeDtypeStruct((2, NP), jnp.float32),
        ],
    )(xp, W, av)


def _sc_edge(xt_pad, a_dst, a_src, pki):
    mesh = plsc.VectorSubcoreMesh(core_axis_name="c", subcore_axis_name="s")
    cp = pltpu.CompilerParams()
    if "needs_layout_passes" in pltpu.CompilerParams.__dataclass_fields__:
        cp = dataclasses.replace(cp, needs_layout_passes=False)

    @functools.partial(
        pl.kernel,
        compiler_params=cp,
        out_type=[
            jax.ShapeDtypeStruct((NC, NP, D), jnp.float32),
            jax.ShapeDtypeStruct((NC * NS, NP), jnp.float32),
        ],
        mesh=mesh,
        scratch_types=[
            pltpu.VMEM((NP,), jnp.float32),
            pltpu.VMEM((NP,), jnp.float32),
            pltpu.VMEM((NP,), jnp.float32),
            pltpu.VMEM((G,), jnp.int32),
            pltpu.VMEM((G,), jnp.int32),
            pltpu.VMEM((G,), jnp.int32),
            pltpu.VMEM((G,), jnp.int32),
            pltpu.VMEM((G,), jnp.int32),
            pltpu.VMEM((G,), jnp.int32),
            pltpu.VMEM((G, D), jnp.float32),
            pltpu.VMEM((G, D), jnp.float32),
            pltpu.VMEM((G,), jnp.float32),
            pltpu.VMEM_SHARED((NP, D), jnp.float32),
            pltpu.SemaphoreType.DMA,
            pltpu.SemaphoreType.DMA,
            pltpu.SemaphoreType.DMA,
            pltpu.SemaphoreType.DMA,
            pltpu.SemaphoreType.DMA,
            pltpu.SemaphoreType.DMA,
        ],
    )
    def k(xt_hbm, adst_hbm, asrc_hbm, pki_hbm, num_hbm, den_hbm,
          adst_v, asrc_v, den_v, pk0, pk1,
          seg_sc0, seg_sc1, col_sc0, col_sc1, rows0, rows1, p_v, acc_sh,
          sem_g0, sem_g1, sem_s0, sem_s1, sem_i0, sem_i1):
        c = lax.axis_index("c")
        s = lax.axis_index("s")
        wid = s * _i32(NC) + c
        z16 = jnp.zeros((L,), jnp.float32)
        pk = (pk0, pk1)
        seg_sc = (seg_sc0, seg_sc1)
        col_sc = (col_sc0, col_sc1)
        rows = (rows0, rows1)
        sem_g = (sem_g0, sem_g1)
        sem_s = (sem_s0, sem_s1)
        sem_i = (sem_i0, sem_i1)

        @pl.loop(_i32(0), _i32(G))
        def _(j):
            for cc in range(D // L):
                rows0[j, pl.ds(cc * L, L)] = z16

        for t in range(RZ // G):
            pltpu.sync_copy(rows0, acc_sh.at[pl.ds(s * _i32(RZ) + _i32(t * G), G)])

        @pl.loop(_i32(0), _i32(NP // L))
        def _(i):
            den_v[pl.ds(i * _i32(L), L)] = z16

        pltpu.sync_copy(adst_hbm, adst_v)
        pltpu.sync_copy(asrc_hbm, asrc_v)

        plsc.subcore_barrier()

        def idx_off(w):
            return wid * _i32(KW) + w * _i32(G)

        def start_idx(w, buf):
            pltpu.async_copy(pki_hbm.at[pl.ds(idx_off(w), G)], pk[buf],
                             sem_i[buf])

        def wait_idx(w, buf):
            pltpu.make_async_copy(pki_hbm.at[pl.ds(idx_off(w), G)], pk[buf],
                                  sem_i[buf]).wait()

        def unpack(buf):
            for v in range(G // L):
                sl = pl.ds(v * L, L)
                w = pk[buf][sl]
                seg_sc[buf][sl] = w & _i32(0xFFFF)
                col_sc[buf][sl] = lax.shift_right_logical(w, _i32(16))

        def start_gather(buf):
            pltpu.async_copy(xt_hbm.at[col_sc[buf]], rows[buf], sem_g[buf])

        def wait_gather(buf):
            pltpu.make_async_copy(xt_hbm.at[col_sc[buf]], rows[buf],
                                  sem_g[buf]).wait()

        def start_scatter(buf):
            pltpu.async_copy(rows[buf], acc_sh.at[seg_sc[buf]], sem_s[buf],
                             add=True)

        def wait_scatter(buf):
            pltpu.make_async_copy(rows[buf], acc_sh.at[seg_sc[buf]],
                                  sem_s[buf]).wait()

        def compute_scale(buf):
            for j8 in range(G // L):
                sl = pl.ds(j8 * L, L)
                sidx = seg_sc[buf][sl]
                cidx = col_sc[buf][sl]
                al = (plsc.load_gather(adst_v, [sidx])
                      + plsc.load_gather(asrc_v, [cidx]))
                al = jnp.where(al > 0, al, al * 0.2)
                p = jnp.exp(al)
                p_v[sl] = p
                plsc.addupdate_scatter(den_v, [sidx], p)

            rv = rows[buf]

            @pl.loop(_i32(0), _i32(G // L))
            def _(j16):
                jb = j16 * _i32(L)
                pvec = p_v[pl.ds(jb, L)]
                for l in range(L):
                    pv = jnp.broadcast_to(pvec[l], (L,))
                    for cc in range(D // L):
                        sl = pl.ds(cc * L, L)
                        rv[jb + _i32(l), sl] = rv[jb + _i32(l), sl] * pv

        pltpu.sync_copy(pki_hbm.at[pl.ds(idx_off(_i32(0)), G)], pk0)
        unpack(0)
        start_gather(0)
        start_idx(_i32(1), 1)
        start_idx(_i32(2), 0)

        @pl.loop(_i32(0), _i32(K // 2))
        def _(i2):
            a = i2 * _i32(2)
            b = a + _i32(1)
            cn = a + _i32(2)

            @pl.when(i2 > _i32(0))
            def _():
                wait_scatter(1)

            wait_idx(b, 1)
            unpack(1)
            start_gather(1)

            @pl.when(b + _i32(2) < _i32(K))
            def _():
                start_idx(b + _i32(2), 1)

            wait_gather(0)
            compute_scale(0)
            start_scatter(0)
            wait_gather(1)
            compute_scale(1)
            wait_scatter(0)

            @pl.when(cn < _i32(K))
            def _():
                wait_idx(cn, 0)
                unpack(0)
                start_gather(0)

                @pl.when(cn + _i32(2) < _i32(K))
                def _():
                    start_idx(cn + _i32(2), 0)

            start_scatter(1)

        wait_scatter(1)

        plsc.subcore_barrier()

        pltpu.sync_copy(acc_sh.at[pl.ds(s * _i32(RZ), RZ)],
                        num_hbm.at[c, pl.ds(s * _i32(RZ), RZ)])
        pltpu.sync_copy(den_v, den_hbm.at[wid])

    return k(xt_pad, a_dst, a_src, pki)


def _combine_body(num_ref, den_ref, bias_ref, out_ref):
    n = num_ref[0] + num_ref[1]
    d = jnp.sum(den_ref[...], axis=0)
    out_ref[...] = n / (d[:, None] + 1e-16) + bias_ref[0][None, :]


def _combine(num, den, bias2d):
    blk = 512
    return pl.pallas_call(
        _combine_body,
        grid=(NP // blk,),
        in_specs=[
            pl.BlockSpec((NC, blk, D), lambda i: (_i32(0), i, _i32(0))),
            pl.BlockSpec((NC * NS, blk), lambda i: (_i32(0), i)),
            pl.BlockSpec((1, D), lambda i: (_i32(0), _i32(0))),
        ],
        out_specs=pl.BlockSpec((blk, D), lambda i: (i, _i32(0))),
        out_shape=jax.ShapeDtypeStruct((NP, D), jnp.float32),
    )(num, den, bias2d)


def kernel(x, edge_index, W, att, bias):
    N = x.shape[0]
    E = edge_index.shape[1]
    xp = jnp.zeros((NP, D), jnp.float32).at[:N].set(x.astype(jnp.float32))
    att_f = att.reshape(2 * D).astype(jnp.float32)
    av = jnp.stack([att_f[:D], att_f[D:]], axis=1)

    xt_pad, a2 = _linear(xp, W.astype(jnp.float32), av)
    a_dst, a_src = a2[0], a2[1]

    row = edge_index[0].astype(jnp.int32)
    col = edge_index[1].astype(jnp.int32)
    loop = jnp.arange(N, dtype=jnp.int32)
    pad = EP - E - N
    seg = jnp.concatenate([
        jnp.where(row != col, row, N), loop,
        jnp.full((pad,), N, jnp.int32),
    ])
    colg = jnp.concatenate([col, loop, jnp.zeros((pad,), jnp.int32)])
    pki = seg | (colg << 16)

    num, den = _sc_edge(xt_pad, a_dst, a_src, pki)
    out = _combine(num, den, bias.astype(jnp.float32).reshape(1, D))
    return out[:N]
